# Initial kernel scaffold; baseline (speedup 1.0000x reference)
import functools

import jax
import jax.numpy as jnp
from jax import lax
from jax.experimental import pallas as pl
from jax.experimental.pallas import tpu as pltpu
from jax.experimental.pallas import tpu_sc as plsc

N = 10000
E = 320000
F_IN = 128
F_HID = 128
F_OUT = 64

NC = 2
NS = 16
NW = NC * NS
EPT = E // NW
CH = 80
NCH = EPT // CH
RPT = N // NS


def _make_sc_agg(D, with_deg):
    out_type = jax.ShapeDtypeStruct((NC, N, D), jnp.float32)
    if with_deg:
        out_type = (out_type, jax.ShapeDtypeStruct((NC, N), jnp.float32))
    scratch = [
        pltpu.VMEM_SHARED((N, D), jnp.float32),
        pltpu.VMEM((CH,), jnp.int32),
        pltpu.VMEM((CH,), jnp.int32),
        pltpu.VMEM((CH, D), jnp.float32),
        pltpu.SemaphoreType.DMA,
    ]
    if with_deg:
        scratch += [
            pltpu.VMEM_SHARED((N,), jnp.float32),
            pltpu.VMEM((CH,), jnp.float32),
        ]

    def body(feat, src, dst, z2d, z1d, *refs):
        if with_deg:
            out, deg_out = refs[0], refs[1]
            acc_sh, src_v, dst_v, rows_v, sem, deg_sh, ones_v = refs[2:]
        else:
            out = refs[0]
            acc_sh, src_v, dst_v, rows_v, sem = refs[1:]
        c = lax.axis_index("c")
        s = lax.axis_index("s")
        wid = c * NS + s
        r0 = pl.multiple_of(s * RPT, 8)
        pltpu.sync_copy(z2d.at[pl.ds(r0, RPT)], acc_sh.at[pl.ds(r0, RPT)])
        if with_deg:
            pltpu.sync_copy(z1d.at[pl.ds(r0, RPT)], deg_sh.at[pl.ds(r0, RPT)])
            for j in range(CH // 16):
                ones_v[pl.ds(j * 16, 16)] = jnp.ones((16,), jnp.float32)
        plsc.subcore_barrier()

        def step(i, carry):
            base = pl.multiple_of(wid * EPT + i * CH, 8)
            pltpu.sync_copy(src.at[pl.ds(base, CH)], src_v)
            pltpu.sync_copy(dst.at[pl.ds(base, CH)], dst_v)
            pltpu.async_copy(feat.at[src_v], rows_v, sem).wait()
            pltpu.sync_copy(rows_v, acc_sh.at[dst_v], add=True)
            if with_deg:
                pltpu.sync_copy(ones_v, deg_sh.at[dst_v], add=True)
            return carry

        lax.fori_loop(0, NCH, step, 0)
        plsc.subcore_barrier()
        pltpu.sync_copy(acc_sh.at[pl.ds(r0, RPT)], out.at[c, pl.ds(r0, RPT)])
        if with_deg:
            pltpu.sync_copy(deg_sh.at[pl.ds(r0, RPT)], deg_out.at[c, pl.ds(r0, RPT)])

    return pl.kernel(
        body,
        out_type=out_type,
        mesh=plsc.VectorSubcoreMesh(core_axis_name="c", subcore_axis_name="s"),
        scratch_types=scratch,
        name=f"sc_agg_d{D}" + ("_deg" if with_deg else ""),
    )


_sc_agg_deg = _make_sc_agg(F_HID, with_deg=True)
_sc_agg = _make_sc_agg(F_OUT, with_deg=False)


_BR = 1000


def _mm1_body(x_ref, ws_ref, wn_ref, b_ref, os_ref, on_ref):
    xb = x_ref[...]
    os_ref[...] = jnp.dot(xb, ws_ref[...], preferred_element_type=jnp.float32) + b_ref[...]
    on_ref[...] = jnp.dot(xb, wn_ref[...], preferred_element_type=jnp.float32)


def _tc_mm1(x, W_self, W_neigh, b):
    grid = (N // _BR,)
    return pl.pallas_call(
        _mm1_body,
        grid=grid,
        in_specs=[
            pl.BlockSpec((_BR, F_IN), lambda i: (i, 0)),
            pl.BlockSpec((F_IN, F_HID), lambda i: (0, 0)),
            pl.BlockSpec((F_IN, F_HID), lambda i: (0, 0)),
            pl.BlockSpec((1, F_HID), lambda i: (0, 0)),
        ],
        out_specs=[
            pl.BlockSpec((_BR, F_HID), lambda i: (i, 0)),
            pl.BlockSpec((_BR, F_HID), lambda i: (i, 0)),
        ],
        out_shape=[
            jax.ShapeDtypeStruct((N, F_HID), jnp.float32),
            jax.ShapeDtypeStruct((N, F_HID), jnp.float32),
        ],
    )(x, W_self, W_neigh, b.reshape(1, -1))


def _l2_body(xs_ref, aggA_ref, aggB_ref, degA_ref, degB_ref, ws_ref, wn_ref,
             b_ref, hs_ref, hn_ref):
    deg = jnp.maximum(degA_ref[...] + degB_ref[...], 1.0)
    h = jnp.maximum(xs_ref[...] + (aggA_ref[...] + aggB_ref[...]) / deg, 0.0)
    hs_ref[...] = jnp.dot(h, ws_ref[...], preferred_element_type=jnp.float32) + b_ref[...]
    hn_ref[...] = jnp.dot(h, wn_ref[...], preferred_element_type=jnp.float32)


def _tc_layer2(xs, aggA, aggB, degA, degB, W_self, W_neigh, b):
    grid = (N // _BR,)
    return pl.pallas_call(
        _l2_body,
        grid=grid,
        in_specs=[
            pl.BlockSpec((_BR, F_HID), lambda i: (i, 0)),
            pl.BlockSpec((_BR, F_HID), lambda i: (i, 0)),
            pl.BlockSpec((_BR, F_HID), lambda i: (i, 0)),
            pl.BlockSpec((_BR, 1), lambda i: (i, 0)),
            pl.BlockSpec((_BR, 1), lambda i: (i, 0)),
            pl.BlockSpec((F_HID, F_OUT), lambda i: (0, 0)),
            pl.BlockSpec((F_HID, F_OUT), lambda i: (0, 0)),
            pl.BlockSpec((1, F_OUT), lambda i: (0, 0)),
        ],
        out_specs=[
            pl.BlockSpec((_BR, F_OUT), lambda i: (i, 0)),
            pl.BlockSpec((_BR, F_OUT), lambda i: (i, 0)),
        ],
        out_shape=[
            jax.ShapeDtypeStruct((N, F_OUT), jnp.float32),
            jax.ShapeDtypeStruct((N, F_OUT), jnp.float32),
        ],
    )(xs, aggA, aggB, degA.reshape(-1, 1), degB.reshape(-1, 1),
      W_self, W_neigh, b.reshape(1, -1))


def _comb_body(hs_ref, aggA_ref, aggB_ref, degA_ref, degB_ref, out_ref):
    deg = jnp.maximum(degA_ref[...] + degB_ref[...], 1.0)
    out_ref[...] = hs_ref[...] + (aggA_ref[...] + aggB_ref[...]) / deg


def _tc_combine(hs, aggA, aggB, degA, degB):
    grid = (N // _BR,)
    return pl.pallas_call(
        _comb_body,
        grid=grid,
        in_specs=[
            pl.BlockSpec((_BR, F_OUT), lambda i: (i, 0)),
            pl.BlockSpec((_BR, F_OUT), lambda i: (i, 0)),
            pl.BlockSpec((_BR, F_OUT), lambda i: (i, 0)),
            pl.BlockSpec((_BR, 1), lambda i: (i, 0)),
            pl.BlockSpec((_BR, 1), lambda i: (i, 0)),
        ],
        out_specs=pl.BlockSpec((_BR, F_OUT), lambda i: (i, 0)),
        out_shape=jax.ShapeDtypeStruct((N, F_OUT), jnp.float32),
    )(hs, aggA, aggB, degA.reshape(-1, 1), degB.reshape(-1, 1))


def kernel(x, edge_index, W_self1, W_neigh1, b1, W_self2, W_neigh2, b2):
    src = edge_index[0].astype(jnp.int32)
    dst = edge_index[1].astype(jnp.int32)
    z2d = jnp.zeros((N, F_HID), jnp.float32)
    z1d = jnp.zeros((N,), jnp.float32)

    xs1, xn1 = _tc_mm1(x, W_self1, W_neigh1, b1)
    agg1, deg = _sc_agg_deg(xn1, src, dst, z2d, z1d)
    hs2, hn2 = _tc_layer2(xs1, agg1[0], agg1[1], deg[0], deg[1],
                          W_self2, W_neigh2, b2)
    agg2 = _sc_agg(hn2, src, dst, z2d[:, :F_OUT], z1d)
    return _tc_combine(hs2, agg2[0], agg2[1], deg[0], deg[1])


# speedup vs baseline: 5.2827x; 5.2827x over previous
import functools

import jax
import jax.numpy as jnp
from jax import lax
from jax.experimental import pallas as pl
from jax.experimental.pallas import tpu as pltpu
from jax.experimental.pallas import tpu_sc as plsc

N = 10000
E = 320000
F_IN = 128
F_HID = 128
F_OUT = 64

NC = 2
NS = 16
NW = NC * NS
EPT = E // NW
CH = 80
NCH = EPT // CH
RPT = 624
R_TAIL0 = NS * RPT
R_TAIL = N - R_TAIL0


def _make_sc_agg(D, with_deg):
    out_type = jax.ShapeDtypeStruct((NC, N, D), jnp.float32)
    if with_deg:
        out_type = (out_type, jax.ShapeDtypeStruct((NC * N,), jnp.float32))
    scratch = [
        pltpu.VMEM_SHARED((N, D), jnp.float32),
        pltpu.VMEM((CH,), jnp.int32),
        pltpu.VMEM((CH,), jnp.int32),
        pltpu.VMEM((CH, D), jnp.float32),
        pltpu.SemaphoreType.DMA,
    ]
    if with_deg:
        scratch += [
            pltpu.VMEM_SHARED((N,), jnp.float32),
            pltpu.VMEM((CH,), jnp.float32),
            pltpu.VMEM((RPT,), jnp.float32),
        ]

    def body(feat, src, dst, *refs):
        if with_deg:
            out, deg_out = refs[0], refs[1]
            acc_sh, src_v, dst_v, rows_v, sem, deg_sh, ones_v, degrow_v = refs[2:]
        else:
            out = refs[0]
            acc_sh, src_v, dst_v, rows_v, sem = refs[1:]
        c = lax.axis_index("c")
        s = lax.axis_index("s")
        wid = c * NS + s
        r0 = pl.multiple_of(s * RPT, 8)

        zv = jnp.zeros((16,), jnp.float32)

        def zrow(i, carry):
            for j in range(D // 16):
                rows_v[i, pl.ds(j * 16, 16)] = zv
            return carry

        lax.fori_loop(0, CH, zrow, 0)
        for k in range(RPT // CH):
            pltpu.sync_copy(rows_v, acc_sh.at[pl.ds(r0 + k * CH, CH)])
        rem = RPT - (RPT // CH) * CH
        pltpu.sync_copy(rows_v.at[pl.ds(0, rem)],
                        acc_sh.at[pl.ds(r0 + RPT - rem, rem)])
        if with_deg:
            for j in range(CH // 16):
                ones_v[pl.ds(j * 16, 16)] = jnp.ones((16,), jnp.float32)
            for j in range(RPT // 16):
                degrow_v[pl.ds(j * 16, 16)] = zv
            pltpu.sync_copy(degrow_v, deg_sh.at[pl.ds(r0, RPT)])

        @pl.when(s == NS - 1)
        def _zero_tail():
            pltpu.sync_copy(rows_v.at[pl.ds(0, R_TAIL)],
                            acc_sh.at[pl.ds(R_TAIL0, R_TAIL)])
            if with_deg:
                pltpu.sync_copy(degrow_v.at[pl.ds(0, R_TAIL)],
                                deg_sh.at[pl.ds(R_TAIL0, R_TAIL)])

        plsc.subcore_barrier()

        def step(i, carry):
            base = pl.multiple_of(wid * EPT + i * CH, 8)
            pltpu.sync_copy(src.at[pl.ds(base, CH)], src_v)
            pltpu.sync_copy(dst.at[pl.ds(base, CH)], dst_v)
            pltpu.async_copy(feat.at[src_v], rows_v, sem).wait()
            pltpu.sync_copy(rows_v, acc_sh.at[dst_v], add=True)
            if with_deg:
                pltpu.sync_copy(ones_v, deg_sh.at[dst_v], add=True)
            return carry

        lax.fori_loop(0, NCH, step, 0)
        plsc.subcore_barrier()

        def wr_rows(lo, nrows):
            pltpu.sync_copy(acc_sh.at[pl.ds(lo, nrows)],
                            rows_v.at[pl.ds(0, nrows)])
            pltpu.sync_copy(rows_v.at[pl.ds(0, nrows)],
                            out.at[c, pl.ds(lo, nrows)])

        for k in range(RPT // CH):
            wr_rows(pl.multiple_of(r0 + k * CH, 8), CH)
        wr_rows(pl.multiple_of(r0 + RPT - rem, 8), rem)
        if with_deg:
            pltpu.sync_copy(deg_sh.at[pl.ds(r0, RPT)], degrow_v)
            d0 = pl.multiple_of(c * N + r0, 8)
            pltpu.sync_copy(degrow_v, deg_out.at[pl.ds(d0, RPT)])

        @pl.when(s == NS - 1)
        def _write_tail():
            wr_rows(R_TAIL0, R_TAIL)
            if with_deg:
                pltpu.sync_copy(deg_sh.at[pl.ds(R_TAIL0, R_TAIL)],
                                degrow_v.at[pl.ds(0, R_TAIL)])
                dt = pl.multiple_of(c * N + R_TAIL0, 8)
                pltpu.sync_copy(degrow_v.at[pl.ds(0, R_TAIL)],
                                deg_out.at[pl.ds(dt, R_TAIL)])

    return pl.kernel(
        body,
        out_type=out_type,
        mesh=plsc.VectorSubcoreMesh(core_axis_name="c", subcore_axis_name="s"),
        scratch_types=scratch,
        name=f"sc_agg_d{D}" + ("_deg" if with_deg else ""),
    )


_sc_agg_deg = _make_sc_agg(F_HID, with_deg=True)
_sc_agg = _make_sc_agg(F_HID, with_deg=False)


_BR = 1000


def _mm1_body(x_ref, ws_ref, wn_ref, b_ref, os_ref, on_ref):
    xb = x_ref[...]
    os_ref[...] = jnp.dot(xb, ws_ref[...], preferred_element_type=jnp.float32) + b_ref[...]
    on_ref[...] = jnp.dot(xb, wn_ref[...], preferred_element_type=jnp.float32)


def _tc_mm1(x, W_self, W_neigh, b):
    grid = (N // _BR,)
    return pl.pallas_call(
        _mm1_body,
        grid=grid,
        in_specs=[
            pl.BlockSpec((_BR, F_IN), lambda i: (i, 0)),
            pl.BlockSpec((F_IN, F_HID), lambda i: (0, 0)),
            pl.BlockSpec((F_IN, F_HID), lambda i: (0, 0)),
            pl.BlockSpec((1, F_HID), lambda i: (0, 0)),
        ],
        out_specs=[
            pl.BlockSpec((_BR, F_HID), lambda i: (i, 0)),
            pl.BlockSpec((_BR, F_HID), lambda i: (i, 0)),
        ],
        out_shape=[
            jax.ShapeDtypeStruct((N, F_HID), jnp.float32),
            jax.ShapeDtypeStruct((N, F_HID), jnp.float32),
        ],
    )(x, W_self, W_neigh, b.reshape(1, -1))


def _l2_body(xs_ref, aggA_ref, aggB_ref, degA_ref, degB_ref, ws_ref,
             b_ref, hs_ref, h_ref):
    deg = jnp.maximum(degA_ref[...] + degB_ref[...], 1.0)
    h = jnp.maximum(xs_ref[...] + (aggA_ref[...] + aggB_ref[...]) / deg, 0.0)
    h_ref[...] = h
    hs_ref[...] = jnp.dot(h, ws_ref[...], preferred_element_type=jnp.float32) + b_ref[...]


def _tc_layer2(xs, aggA, aggB, degA, degB, W_self, b):
    grid = (N // _BR,)
    return pl.pallas_call(
        _l2_body,
        grid=grid,
        in_specs=[
            pl.BlockSpec((_BR, F_HID), lambda i: (i, 0)),
            pl.BlockSpec((_BR, F_HID), lambda i: (i, 0)),
            pl.BlockSpec((_BR, F_HID), lambda i: (i, 0)),
            pl.BlockSpec((_BR, 1), lambda i: (i, 0)),
            pl.BlockSpec((_BR, 1), lambda i: (i, 0)),
            pl.BlockSpec((F_HID, F_OUT), lambda i: (0, 0)),
            pl.BlockSpec((1, F_OUT), lambda i: (0, 0)),
        ],
        out_specs=[
            pl.BlockSpec((_BR, F_OUT), lambda i: (i, 0)),
            pl.BlockSpec((_BR, F_HID), lambda i: (i, 0)),
        ],
        out_shape=[
            jax.ShapeDtypeStruct((N, F_OUT), jnp.float32),
            jax.ShapeDtypeStruct((N, F_HID), jnp.float32),
        ],
    )(xs, aggA, aggB, degA.reshape(-1, 1), degB.reshape(-1, 1),
      W_self, b.reshape(1, -1))


def _comb_body(hs_ref, aggA_ref, aggB_ref, degA_ref, degB_ref, wn_ref, out_ref):
    deg = jnp.maximum(degA_ref[...] + degB_ref[...], 1.0)
    h_neigh = (aggA_ref[...] + aggB_ref[...]) / deg
    out_ref[...] = hs_ref[...] + jnp.dot(
        h_neigh, wn_ref[...], preferred_element_type=jnp.float32)


def _tc_combine(hs, aggA, aggB, degA, degB, W_neigh):
    grid = (N // _BR,)
    return pl.pallas_call(
        _comb_body,
        grid=grid,
        in_specs=[
            pl.BlockSpec((_BR, F_OUT), lambda i: (i, 0)),
            pl.BlockSpec((_BR, F_HID), lambda i: (i, 0)),
            pl.BlockSpec((_BR, F_HID), lambda i: (i, 0)),
            pl.BlockSpec((_BR, 1), lambda i: (i, 0)),
            pl.BlockSpec((_BR, 1), lambda i: (i, 0)),
            pl.BlockSpec((F_HID, F_OUT), lambda i: (0, 0)),
        ],
        out_specs=pl.BlockSpec((_BR, F_OUT), lambda i: (i, 0)),
        out_shape=jax.ShapeDtypeStruct((N, F_OUT), jnp.float32),
    )(hs, aggA, aggB, degA.reshape(-1, 1), degB.reshape(-1, 1), W_neigh)


def kernel(x, edge_index, W_self1, W_neigh1, b1, W_self2, W_neigh2, b2):
    src = edge_index[0].astype(jnp.int32)
    dst = edge_index[1].astype(jnp.int32)

    xs1, xn1 = _tc_mm1(x, W_self1, W_neigh1, b1)
    agg1, deg = _sc_agg_deg(xn1, src, dst)
    degA, degB = deg[:N], deg[N:]
    hs2, h = _tc_layer2(xs1, agg1[0], agg1[1], degA, degB, W_self2, b2)
    agg2 = _sc_agg(h, src, dst)
    return _tc_combine(hs2, agg2[0], agg2[1], degA, degB, W_neigh2)


# speedup vs baseline: 9.8988x; 1.8738x over previous
import functools

import jax
import jax.numpy as jnp
from jax import lax
from jax.experimental import pallas as pl
from jax.experimental.pallas import tpu as pltpu
from jax.experimental.pallas import tpu_sc as plsc

N = 10000
E = 320000
F_IN = 128
F_HID = 128
F_OUT = 64

NC = 2
NS = 16
NW = NC * NS
EPT = E // NW
CH = 80
NCH = EPT // CH
RPT = 624
R_TAIL0 = NS * RPT
R_TAIL = N - R_TAIL0


def _make_sc_agg(D, with_deg):
    out_type = jax.ShapeDtypeStruct((NC, N, D), jnp.float32)
    if with_deg:
        out_type = (out_type, jax.ShapeDtypeStruct((NC * N,), jnp.float32))
    scratch = [
        pltpu.VMEM_SHARED((N, D), jnp.float32),
        pltpu.VMEM((EPT,), jnp.int32),
        pltpu.VMEM((CH,), jnp.int32),
        pltpu.VMEM((CH,), jnp.int32),
        pltpu.VMEM((CH, D), jnp.float32),
        pltpu.VMEM((CH, D), jnp.float32),
        pltpu.SemaphoreType.DMA,
        pltpu.SemaphoreType.DMA,
        pltpu.SemaphoreType.DMA,
        pltpu.SemaphoreType.DMA,
        pltpu.SemaphoreType.DMA,
        pltpu.SemaphoreType.DMA,
    ]
    if with_deg:
        scratch += [
            pltpu.VMEM_SHARED((N,), jnp.float32),
            pltpu.VMEM((CH,), jnp.float32),
            pltpu.VMEM((RPT,), jnp.float32),
            pltpu.SemaphoreType.DMA,
            pltpu.SemaphoreType.DMA,
        ]

    LOOP_CHUNKS = NCH - 1
    NPAIR = LOOP_CHUNKS // 2

    def body(feat, src, dst, *refs):
        if with_deg:
            out, deg_out = refs[0], refs[1]
            (acc_sh, src_all, dst_v0, dst_v1, rows0, rows1,
             isem0, isem1, gsem0, gsem1, ssem0, ssem1,
             deg_sh, ones_v, degrow_v, dsem0, dsem1) = refs[2:]
        else:
            out = refs[0]
            (acc_sh, src_all, dst_v0, dst_v1, rows0, rows1,
             isem0, isem1, gsem0, gsem1, ssem0, ssem1) = refs[1:]
        c = lax.axis_index("c")
        s = lax.axis_index("s")
        wid = c * NS + s
        r0 = pl.multiple_of(s * RPT, 8)
        ebase = pl.multiple_of(wid * EPT, 8)

        zv = jnp.zeros((16,), jnp.float32)

        def zrow(i, carry):
            for j in range(D // 16):
                rows0[i, pl.ds(j * 16, 16)] = zv
            return carry

        lax.fori_loop(0, CH, zrow, 0)
        for k in range(RPT // CH):
            pltpu.sync_copy(rows0, acc_sh.at[pl.ds(r0 + k * CH, CH)])
        rem = RPT - (RPT // CH) * CH
        pltpu.sync_copy(rows0.at[pl.ds(0, rem)],
                        acc_sh.at[pl.ds(r0 + RPT - rem, rem)])
        if with_deg:
            for j in range(CH // 16):
                ones_v[pl.ds(j * 16, 16)] = jnp.ones((16,), jnp.float32)
            for j in range(RPT // 16):
                degrow_v[pl.ds(j * 16, 16)] = zv
            pltpu.sync_copy(degrow_v, deg_sh.at[pl.ds(r0, RPT)])

        @pl.when(s == NS - 1)
        def _zero_tail():
            pltpu.sync_copy(rows0.at[pl.ds(0, R_TAIL)],
                            acc_sh.at[pl.ds(R_TAIL0, R_TAIL)])
            if with_deg:
                pltpu.sync_copy(degrow_v.at[pl.ds(0, R_TAIL)],
                                deg_sh.at[pl.ds(R_TAIL0, R_TAIL)])

        plsc.subcore_barrier()

        pltpu.sync_copy(src.at[pl.ds(ebase, EPT)], src_all)

        def src_sl(i):
            return src_all.at[pl.ds(pl.multiple_of(i * CH, 8), CH)]

        def start_chunk(i, dst_v, rows, isem, gsem):
            pltpu.async_copy(dst.at[pl.ds(pl.multiple_of(ebase + i * CH, 8), CH)],
                             dst_v, isem)
            pltpu.async_copy(feat.at[src_sl(i)], rows, gsem)

        def start_scatter(dst_v, rows, ssem, dsem):
            pltpu.async_copy(rows, acc_sh.at[dst_v], ssem, add=True)
            if with_deg:
                pltpu.async_copy(ones_v, deg_sh.at[dst_v], dsem, add=True)

        def wait_idx(dst_v, isem):
            pltpu.make_async_copy(dst.at[pl.ds(ebase, CH)], dst_v, isem).wait()

        def wait_gather(rows, gsem):
            pltpu.make_async_copy(feat.at[src_sl(0)], rows, gsem).wait()

        def wait_scatter(dst_v, rows, ssem, dsem):
            pltpu.make_async_copy(rows, acc_sh.at[dst_v], ssem).wait()
            if with_deg:
                pltpu.make_async_copy(ones_v, deg_sh.at[dst_v], dsem).wait()

        start_chunk(LOOP_CHUNKS, dst_v0, rows0, isem0, gsem0)
        wait_idx(dst_v0, isem0)
        wait_gather(rows0, gsem0)
        start_scatter(dst_v0, rows0, ssem0, dsem0 if with_deg else None)
        wait_scatter(dst_v0, rows0, ssem0, dsem0 if with_deg else None)

        start_chunk(0, dst_v0, rows0, isem0, gsem0)
        start_chunk(1, dst_v1, rows1, isem1, gsem1)

        def pair(g, carry):
            i0 = 2 * g
            i1 = i0 + 1
            wait_gather(rows0, gsem0)
            wait_idx(dst_v0, isem0)
            start_scatter(dst_v0, rows0, ssem0, dsem0 if with_deg else None)
            wait_gather(rows1, gsem1)
            wait_idx(dst_v1, isem1)
            start_scatter(dst_v1, rows1, ssem1, dsem1 if with_deg else None)
            n0 = jnp.where(i0 + 2 < LOOP_CHUNKS, i0 + 2, 0)
            wait_scatter(dst_v0, rows0, ssem0, dsem0 if with_deg else None)
            start_chunk(n0, dst_v0, rows0, isem0, gsem0)
            n1 = jnp.where(i1 + 2 < LOOP_CHUNKS, i1 + 2, 0)
            wait_scatter(dst_v1, rows1, ssem1, dsem1 if with_deg else None)
            start_chunk(n1, dst_v1, rows1, isem1, gsem1)
            return carry

        lax.fori_loop(0, NPAIR, pair, 0)
        wait_idx(dst_v0, isem0)
        wait_gather(rows0, gsem0)
        wait_idx(dst_v1, isem1)
        wait_gather(rows1, gsem1)
        plsc.subcore_barrier()

        def wr_start(lo, nrows, rows, ssem):
            pltpu.sync_copy(acc_sh.at[pl.ds(lo, nrows)],
                            rows.at[pl.ds(0, nrows)])
            pltpu.async_copy(rows.at[pl.ds(0, nrows)],
                             out.at[c, pl.ds(lo, nrows)], ssem)

        def wr_wait(lo, nrows, rows, ssem):
            pltpu.make_async_copy(rows.at[pl.ds(0, nrows)],
                                  out.at[c, pl.ds(lo, nrows)], ssem).wait()

        nwr = RPT // CH
        for k in range(nwr):
            lo = pl.multiple_of(r0 + k * CH, 8)
            rows, ssem = (rows0, ssem0) if k % 2 == 0 else (rows1, ssem1)
            if k >= 2:
                wr_wait(lo, CH, rows, ssem)
            wr_start(lo, CH, rows, ssem)
        lo_rem = pl.multiple_of(r0 + RPT - rem, 8)
        wr_wait(lo_rem, CH, rows1, ssem1)
        wr_start(lo_rem, rem, rows1, ssem1)
        wr_wait(lo_rem, CH, rows0, ssem0)
        wr_wait(lo_rem, rem, rows1, ssem1)
        if with_deg:
            pltpu.sync_copy(deg_sh.at[pl.ds(r0, RPT)], degrow_v)
            d0 = pl.multiple_of(c * N + r0, 8)
            pltpu.sync_copy(degrow_v, deg_out.at[pl.ds(d0, RPT)])

        @pl.when(s == NS - 1)
        def _write_tail():
            pltpu.sync_copy(acc_sh.at[pl.ds(R_TAIL0, R_TAIL)],
                            rows0.at[pl.ds(0, R_TAIL)])
            pltpu.sync_copy(rows0.at[pl.ds(0, R_TAIL)],
                            out.at[c, pl.ds(R_TAIL0, R_TAIL)])
            if with_deg:
                pltpu.sync_copy(deg_sh.at[pl.ds(R_TAIL0, R_TAIL)],
                                degrow_v.at[pl.ds(0, R_TAIL)])
                dt = pl.multiple_of(c * N + R_TAIL0, 8)
                pltpu.sync_copy(degrow_v.at[pl.ds(0, R_TAIL)],
                                deg_out.at[pl.ds(dt, R_TAIL)])

    return pl.kernel(
        body,
        out_type=out_type,
        mesh=plsc.VectorSubcoreMesh(core_axis_name="c", subcore_axis_name="s"),
        scratch_types=scratch,
        name=f"sc_agg_d{D}" + ("_deg" if with_deg else ""),
    )


_sc_agg_deg = _make_sc_agg(F_HID, with_deg=True)
_sc_agg = _make_sc_agg(F_HID, with_deg=False)


_BR = 1000


def _mm1_body(x_ref, ws_ref, wn_ref, b_ref, os_ref, on_ref):
    xb = x_ref[...]
    os_ref[...] = jnp.dot(xb, ws_ref[...], preferred_element_type=jnp.float32) + b_ref[...]
    on_ref[...] = jnp.dot(xb, wn_ref[...], preferred_element_type=jnp.float32)


def _tc_mm1(x, W_self, W_neigh, b):
    grid = (N // _BR,)
    return pl.pallas_call(
        _mm1_body,
        grid=grid,
        in_specs=[
            pl.BlockSpec((_BR, F_IN), lambda i: (i, 0)),
            pl.BlockSpec((F_IN, F_HID), lambda i: (0, 0)),
            pl.BlockSpec((F_IN, F_HID), lambda i: (0, 0)),
            pl.BlockSpec((1, F_HID), lambda i: (0, 0)),
        ],
        out_specs=[
            pl.BlockSpec((_BR, F_HID), lambda i: (i, 0)),
            pl.BlockSpec((_BR, F_HID), lambda i: (i, 0)),
        ],
        out_shape=[
            jax.ShapeDtypeStruct((N, F_HID), jnp.float32),
            jax.ShapeDtypeStruct((N, F_HID), jnp.float32),
        ],
    )(x, W_self, W_neigh, b.reshape(1, -1))


def _l2_body(xs_ref, aggA_ref, aggB_ref, degA_ref, degB_ref, ws_ref,
             b_ref, hs_ref, h_ref):
    deg = jnp.maximum(degA_ref[...] + degB_ref[...], 1.0)
    h = jnp.maximum(xs_ref[...] + (aggA_ref[...] + aggB_ref[...]) / deg, 0.0)
    h_ref[...] = h
    hs_ref[...] = jnp.dot(h, ws_ref[...], preferred_element_type=jnp.float32) + b_ref[...]


def _tc_layer2(xs, aggA, aggB, degA, degB, W_self, b):
    grid = (N // _BR,)
    return pl.pallas_call(
        _l2_body,
        grid=grid,
        in_specs=[
            pl.BlockSpec((_BR, F_HID), lambda i: (i, 0)),
            pl.BlockSpec((_BR, F_HID), lambda i: (i, 0)),
            pl.BlockSpec((_BR, F_HID), lambda i: (i, 0)),
            pl.BlockSpec((_BR, 1), lambda i: (i, 0)),
            pl.BlockSpec((_BR, 1), lambda i: (i, 0)),
            pl.BlockSpec((F_HID, F_OUT), lambda i: (0, 0)),
            pl.BlockSpec((1, F_OUT), lambda i: (0, 0)),
        ],
        out_specs=[
            pl.BlockSpec((_BR, F_OUT), lambda i: (i, 0)),
            pl.BlockSpec((_BR, F_HID), lambda i: (i, 0)),
        ],
        out_shape=[
            jax.ShapeDtypeStruct((N, F_OUT), jnp.float32),
            jax.ShapeDtypeStruct((N, F_HID), jnp.float32),
        ],
    )(xs, aggA, aggB, degA.reshape(-1, 1), degB.reshape(-1, 1),
      W_self, b.reshape(1, -1))


def _comb_body(hs_ref, aggA_ref, aggB_ref, degA_ref, degB_ref, wn_ref, out_ref):
    deg = jnp.maximum(degA_ref[...] + degB_ref[...], 1.0)
    h_neigh = (aggA_ref[...] + aggB_ref[...]) / deg
    out_ref[...] = hs_ref[...] + jnp.dot(
        h_neigh, wn_ref[...], preferred_element_type=jnp.float32)


def _tc_combine(hs, aggA, aggB, degA, degB, W_neigh):
    grid = (N // _BR,)
    return pl.pallas_call(
        _comb_body,
        grid=grid,
        in_specs=[
            pl.BlockSpec((_BR, F_OUT), lambda i: (i, 0)),
            pl.BlockSpec((_BR, F_HID), lambda i: (i, 0)),
            pl.BlockSpec((_BR, F_HID), lambda i: (i, 0)),
            pl.BlockSpec((_BR, 1), lambda i: (i, 0)),
            pl.BlockSpec((_BR, 1), lambda i: (i, 0)),
            pl.BlockSpec((F_HID, F_OUT), lambda i: (0, 0)),
        ],
        out_specs=pl.BlockSpec((_BR, F_OUT), lambda i: (i, 0)),
        out_shape=jax.ShapeDtypeStruct((N, F_OUT), jnp.float32),
    )(hs, aggA, aggB, degA.reshape(-1, 1), degB.reshape(-1, 1), W_neigh)


def kernel(x, edge_index, W_self1, W_neigh1, b1, W_self2, W_neigh2, b2):
    src = edge_index[0].astype(jnp.int32)
    dst = edge_index[1].astype(jnp.int32)

    xs1, xn1 = _tc_mm1(x, W_self1, W_neigh1, b1)
    agg1, deg = _sc_agg_deg(xn1, src, dst)
    degA, degB = deg[:N], deg[N:]
    hs2, h = _tc_layer2(xs1, agg1[0], agg1[1], degA, degB, W_self2, b2)
    agg2 = _sc_agg(h, src, dst)
    return _tc_combine(hs2, agg2[0], agg2[1], degA, degB, W_neigh2)


# speedup vs baseline: 10.6804x; 1.0790x over previous
import jax
import jax.numpy as jnp
from jax import lax
from jax.experimental import pallas as pl
from jax.experimental.pallas import tpu as pltpu
from jax.experimental.pallas import tpu_sc as plsc

N = 10000
E = 320000
F_IN = 128
F_HID = 128
F_OUT = 64

NC = 2
NS = 16
NW = NC * NS
EPT = E // NW
CH = 128
NFULL = EPT // CH
REM_E = EPT - NFULL * CH
NPAIR = NFULL // 2
RPT = 624
R_TAIL0 = NS * RPT
R_TAIL = N - R_TAIL0
ZFULL = RPT // CH
ZREM = RPT - ZFULL * CH


def _make_sc_agg(D, with_deg):
    out_type = jax.ShapeDtypeStruct((NC, N, D), jnp.float32)
    if with_deg:
        out_type = (out_type, jax.ShapeDtypeStruct((NC * N,), jnp.float32))
    scratch = [
        pltpu.VMEM_SHARED((N, D), jnp.float32),
        pltpu.VMEM((EPT,), jnp.int32),
        pltpu.VMEM((CH,), jnp.int32),
        pltpu.VMEM((CH,), jnp.int32),
        pltpu.VMEM((REM_E,), jnp.int32),
        pltpu.VMEM((CH, D), jnp.float32),
        pltpu.VMEM((CH, D), jnp.float32),
        pltpu.SemaphoreType.DMA,
        pltpu.SemaphoreType.DMA,
        pltpu.SemaphoreType.DMA,
        pltpu.SemaphoreType.DMA,
        pltpu.SemaphoreType.DMA,
        pltpu.SemaphoreType.DMA,
    ]
    if with_deg:
        scratch += [
            pltpu.VMEM_SHARED((N,), jnp.float32),
            pltpu.VMEM((CH,), jnp.float32),
            pltpu.VMEM((RPT,), jnp.float32),
            pltpu.SemaphoreType.DMA,
            pltpu.SemaphoreType.DMA,
        ]

    def body(feat, src, dst, *refs):
        if with_deg:
            out, deg_out = refs[0], refs[1]
            (acc_sh, src_all, dst_v0, dst_v1, dst_rem, rows0, rows1,
             isem0, isem1, gsem0, gsem1, ssem0, ssem1,
             deg_sh, ones_v, degrow_v, dsem0, dsem1) = refs[2:]
        else:
            out = refs[0]
            (acc_sh, src_all, dst_v0, dst_v1, dst_rem, rows0, rows1,
             isem0, isem1, gsem0, gsem1, ssem0, ssem1) = refs[1:]
        c = lax.axis_index("c")
        s = lax.axis_index("s")
        wid = c * NS + s
        r0 = pl.multiple_of(s * RPT, 8)
        ebase = pl.multiple_of(wid * EPT, 8)

        zv = jnp.zeros((16,), jnp.float32)

        def zrow(i, carry):
            for j in range(D // 16):
                rows0[i, pl.ds(j * 16, 16)] = zv
            return carry

        lax.fori_loop(0, CH, zrow, 0)
        for k in range(ZFULL):
            pltpu.sync_copy(rows0, acc_sh.at[pl.ds(r0 + k * CH, CH)])
        pltpu.sync_copy(rows0.at[pl.ds(0, ZREM)],
                        acc_sh.at[pl.ds(r0 + ZFULL * CH, ZREM)])
        if with_deg:
            for j in range(CH // 16):
                ones_v[pl.ds(j * 16, 16)] = jnp.ones((16,), jnp.float32)
            for j in range(RPT // 16):
                degrow_v[pl.ds(j * 16, 16)] = zv
            pltpu.sync_copy(degrow_v, deg_sh.at[pl.ds(r0, RPT)])

        @pl.when(s == NS - 1)
        def _zero_tail():
            pltpu.sync_copy(rows0.at[pl.ds(0, R_TAIL)],
                            acc_sh.at[pl.ds(R_TAIL0, R_TAIL)])
            if with_deg:
                pltpu.sync_copy(degrow_v.at[pl.ds(0, R_TAIL)],
                                deg_sh.at[pl.ds(R_TAIL0, R_TAIL)])

        plsc.subcore_barrier()

        pltpu.sync_copy(src.at[pl.ds(ebase, EPT)], src_all)

        rem_lo = pl.multiple_of(NFULL * CH, 8)
        pltpu.sync_copy(dst.at[pl.ds(ebase + rem_lo, REM_E)], dst_rem)
        pltpu.async_copy(feat.at[src_all.at[pl.ds(rem_lo, REM_E)]],
                         rows0.at[pl.ds(0, REM_E)], gsem0).wait()
        d_rem = pltpu.async_copy(rows0.at[pl.ds(0, REM_E)],
                                 acc_sh.at[dst_rem], ssem0, add=True)
        if with_deg:
            pltpu.async_copy(ones_v.at[pl.ds(0, REM_E)],
                             deg_sh.at[dst_rem], dsem0, add=True).wait()
        d_rem.wait()

        def src_sl(i):
            return src_all.at[pl.ds(pl.multiple_of(i * CH, 8), CH)]

        def start_chunk(i, dst_v, rows, isem, gsem):
            pltpu.async_copy(dst.at[pl.ds(pl.multiple_of(ebase + i * CH, 8), CH)],
                             dst_v, isem)
            pltpu.async_copy(feat.at[src_sl(i)], rows, gsem)

        def start_scatter(dst_v, rows, ssem, dsem):
            pltpu.async_copy(rows, acc_sh.at[dst_v], ssem, add=True)
            if with_deg:
                pltpu.async_copy(ones_v, deg_sh.at[dst_v], dsem, add=True)

        def wait_idx(dst_v, isem):
            pltpu.make_async_copy(dst.at[pl.ds(ebase, CH)], dst_v, isem).wait()

        def wait_gather(rows, gsem):
            pltpu.make_async_copy(feat.at[src_sl(0)], rows, gsem).wait()

        def wait_scatter(dst_v, rows, ssem, dsem):
            pltpu.make_async_copy(rows, acc_sh.at[dst_v], ssem).wait()
            if with_deg:
                pltpu.make_async_copy(ones_v, deg_sh.at[dst_v], dsem).wait()

        start_chunk(0, dst_v0, rows0, isem0, gsem0)
        start_chunk(1, dst_v1, rows1, isem1, gsem1)

        def pair(g, carry):
            i0 = 2 * g
            i1 = i0 + 1
            wait_gather(rows0, gsem0)
            wait_idx(dst_v0, isem0)
            start_scatter(dst_v0, rows0, ssem0, dsem0 if with_deg else None)
            wait_gather(rows1, gsem1)
            wait_idx(dst_v1, isem1)
            start_scatter(dst_v1, rows1, ssem1, dsem1 if with_deg else None)
            n0 = jnp.where(i0 + 2 < NFULL, i0 + 2, 0)
            wait_scatter(dst_v0, rows0, ssem0, dsem0 if with_deg else None)
            start_chunk(n0, dst_v0, rows0, isem0, gsem0)
            n1 = jnp.where(i1 + 2 < NFULL, i1 + 2, 0)
            wait_scatter(dst_v1, rows1, ssem1, dsem1 if with_deg else None)
            start_chunk(n1, dst_v1, rows1, isem1, gsem1)
            return carry

        lax.fori_loop(0, NPAIR, pair, 0)
        wait_idx(dst_v0, isem0)
        wait_gather(rows0, gsem0)
        wait_idx(dst_v1, isem1)
        wait_gather(rows1, gsem1)
        plsc.subcore_barrier()

        def wset(k):
            return (rows0, ssem0) if k % 2 == 0 else (rows1, ssem1)

        def wr_start(lo, nrows, rows, ssem):
            pltpu.sync_copy(acc_sh.at[pl.ds(lo, nrows)],
                            rows.at[pl.ds(0, nrows)])
            pltpu.async_copy(rows.at[pl.ds(0, nrows)],
                             out.at[c, pl.ds(lo, nrows)], ssem)

        def wr_wait(lo, nrows, rows, ssem):
            pltpu.make_async_copy(rows.at[pl.ds(0, nrows)],
                                  out.at[c, pl.ds(lo, nrows)], ssem).wait()

        for k in range(ZFULL):
            lo = pl.multiple_of(r0 + k * CH, 8)
            rows, ssem = wset(k)
            if k >= 2:
                wr_wait(lo, CH, rows, ssem)
            wr_start(lo, CH, rows, ssem)
        lo_rem = pl.multiple_of(r0 + ZFULL * CH, 8)
        rrows, rsem = wset(ZFULL)
        orows, osem = wset(ZFULL + 1)
        wr_wait(lo_rem, CH, rrows, rsem)
        wr_start(lo_rem, ZREM, rrows, rsem)
        wr_wait(lo_rem, CH, orows, osem)
        wr_wait(lo_rem, ZREM, rrows, rsem)
        if with_deg:
            pltpu.sync_copy(deg_sh.at[pl.ds(r0, RPT)], degrow_v)
            d0 = pl.multiple_of(c * N + r0, 8)
            pltpu.sync_copy(degrow_v, deg_out.at[pl.ds(d0, RPT)])

        @pl.when(s == NS - 1)
        def _write_tail():
            pltpu.sync_copy(acc_sh.at[pl.ds(R_TAIL0, R_TAIL)],
                            rows0.at[pl.ds(0, R_TAIL)])
            pltpu.sync_copy(rows0.at[pl.ds(0, R_TAIL)],
                            out.at[c, pl.ds(R_TAIL0, R_TAIL)])
            if with_deg:
                pltpu.sync_copy(deg_sh.at[pl.ds(R_TAIL0, R_TAIL)],
                                degrow_v.at[pl.ds(0, R_TAIL)])
                dt = pl.multiple_of(c * N + R_TAIL0, 8)
                pltpu.sync_copy(degrow_v.at[pl.ds(0, R_TAIL)],
                                deg_out.at[pl.ds(dt, R_TAIL)])

    return pl.kernel(
        body,
        out_type=out_type,
        mesh=plsc.VectorSubcoreMesh(core_axis_name="c", subcore_axis_name="s"),
        scratch_types=scratch,
        name=f"sc_agg_d{D}" + ("_deg" if with_deg else ""),
    )


_sc_agg_deg = _make_sc_agg(F_HID, with_deg=True)
_sc_agg = _make_sc_agg(F_HID, with_deg=False)


_BR = 1000


def _mm1_body(x_ref, ws_ref, wn_ref, b_ref, os_ref, on_ref):
    xb = x_ref[...]
    os_ref[...] = jnp.dot(xb, ws_ref[...], preferred_element_type=jnp.float32) + b_ref[...]
    on_ref[...] = jnp.dot(xb, wn_ref[...], preferred_element_type=jnp.float32)


def _tc_mm1(x, W_self, W_neigh, b):
    return pl.pallas_call(
        _mm1_body,
        grid=(N // _BR,),
        in_specs=[
            pl.BlockSpec((_BR, F_IN), lambda i: (i, 0)),
            pl.BlockSpec((F_IN, F_HID), lambda i: (0, 0)),
            pl.BlockSpec((F_IN, F_HID), lambda i: (0, 0)),
            pl.BlockSpec((1, F_HID), lambda i: (0, 0)),
        ],
        out_specs=[
            pl.BlockSpec((_BR, F_HID), lambda i: (i, 0)),
            pl.BlockSpec((_BR, F_HID), lambda i: (i, 0)),
        ],
        out_shape=[
            jax.ShapeDtypeStruct((N, F_HID), jnp.float32),
            jax.ShapeDtypeStruct((N, F_HID), jnp.float32),
        ],
    )(x, W_self, W_neigh, b.reshape(1, -1))


def _l2_body(xs_ref, agg_ref, deg_ref, ws_ref, b_ref, hs_ref, h_ref):
    deg = jnp.maximum(deg_ref[0] + deg_ref[1], 1.0)
    h = jnp.maximum(xs_ref[...] + (agg_ref[0] + agg_ref[1]) / deg, 0.0)
    h_ref[...] = h
    hs_ref[...] = jnp.dot(h, ws_ref[...], preferred_element_type=jnp.float32) + b_ref[...]


def _tc_layer2(xs, agg, deg2, W_self, b):
    return pl.pallas_call(
        _l2_body,
        grid=(N // _BR,),
        in_specs=[
            pl.BlockSpec((_BR, F_HID), lambda i: (i, 0)),
            pl.BlockSpec((NC, _BR, F_HID), lambda i: (0, i, 0)),
            pl.BlockSpec((NC, _BR, 1), lambda i: (0, i, 0)),
            pl.BlockSpec((F_HID, F_OUT), lambda i: (0, 0)),
            pl.BlockSpec((1, F_OUT), lambda i: (0, 0)),
        ],
        out_specs=[
            pl.BlockSpec((_BR, F_OUT), lambda i: (i, 0)),
            pl.BlockSpec((_BR, F_HID), lambda i: (i, 0)),
        ],
        out_shape=[
            jax.ShapeDtypeStruct((N, F_OUT), jnp.float32),
            jax.ShapeDtypeStruct((N, F_HID), jnp.float32),
        ],
    )(xs, agg, deg2, W_self, b.reshape(1, -1))


def _comb_body(hs_ref, agg_ref, deg_ref, wn_ref, out_ref):
    deg = jnp.maximum(deg_ref[0] + deg_ref[1], 1.0)
    h_neigh = (agg_ref[0] + agg_ref[1]) / deg
    out_ref[...] = hs_ref[...] + jnp.dot(
        h_neigh, wn_ref[...], preferred_element_type=jnp.float32)


def _tc_combine(hs, agg, deg2, W_neigh):
    return pl.pallas_call(
        _comb_body,
        grid=(N // _BR,),
        in_specs=[
            pl.BlockSpec((_BR, F_OUT), lambda i: (i, 0)),
            pl.BlockSpec((NC, _BR, F_HID), lambda i: (0, i, 0)),
            pl.BlockSpec((NC, _BR, 1), lambda i: (0, i, 0)),
            pl.BlockSpec((F_HID, F_OUT), lambda i: (0, 0)),
        ],
        out_specs=pl.BlockSpec((_BR, F_OUT), lambda i: (i, 0)),
        out_shape=jax.ShapeDtypeStruct((N, F_OUT), jnp.float32),
    )(hs, agg, deg2, W_neigh)


def kernel(x, edge_index, W_self1, W_neigh1, b1, W_self2, W_neigh2, b2):
    src = edge_index[0].astype(jnp.int32)
    dst = edge_index[1].astype(jnp.int32)

    xs1, xn1 = _tc_mm1(x, W_self1, W_neigh1, b1)
    agg1, deg = _sc_agg_deg(xn1, src, dst)
    deg2 = deg.reshape(NC, N, 1)
    hs2, h = _tc_layer2(xs1, agg1, deg2, W_self2, b2)
    agg2 = _sc_agg(h, src, dst)
    return _tc_combine(hs2, agg2, deg2, W_neigh2)


# speedup vs baseline: 11.7538x; 1.1005x over previous
import jax
import jax.numpy as jnp
from jax import lax
from jax.experimental import pallas as pl
from jax.experimental.pallas import tpu as pltpu
from jax.experimental.pallas import tpu_sc as plsc

N = 10000
E = 320000
F_IN = 128
F_HID = 128
F_OUT = 64

NC = 2
NS = 16
NW = NC * NS
EPT = E // NW
CH = 96
NFULL = EPT // CH
REM_E = EPT - NFULL * CH
NSETS = 3
NQUAD = NFULL // NSETS
NMAIN = NQUAD * NSETS
RPT = 624
R_TAIL0 = NS * RPT
R_TAIL = N - R_TAIL0
ZFULL = RPT // CH
ZREM = RPT - ZFULL * CH


def _make_sc_agg(D, with_deg):
    out_type = jax.ShapeDtypeStruct((NC, N, D), jnp.float32)
    if with_deg:
        out_type = (out_type, jax.ShapeDtypeStruct((NC * N,), jnp.float32))
    scratch = [
        pltpu.VMEM_SHARED((N, D), jnp.float32),
        pltpu.VMEM((EPT,), jnp.int32),
        pltpu.VMEM((REM_E,), jnp.int32),
    ]
    for _ in range(NSETS):
        scratch += [
            pltpu.VMEM((CH,), jnp.int32),
            pltpu.VMEM((CH, D), jnp.float32),
            pltpu.SemaphoreType.DMA,
            pltpu.SemaphoreType.DMA,
            pltpu.SemaphoreType.DMA,
            pltpu.SemaphoreType.DMA,
        ]
    if with_deg:
        scratch += [
            pltpu.VMEM_SHARED((N,), jnp.float32),
            pltpu.VMEM((CH,), jnp.float32),
            pltpu.VMEM((RPT,), jnp.float32),
        ]

    def body(feat, src, dst, *refs):
        if with_deg:
            out, deg_out = refs[0], refs[1]
            rest = refs[2:]
        else:
            out = refs[0]
            rest = refs[1:]
        acc_sh, src_all, dst_rem = rest[0], rest[1], rest[2]
        sets = [tuple(rest[3 + 6 * q: 3 + 6 * (q + 1)]) for q in range(NSETS)]
        if with_deg:
            deg_sh, ones_v, degrow_v = rest[3 + 6 * NSETS:]
        rows0 = sets[0][1]
        c = lax.axis_index("c")
        s = lax.axis_index("s")
        wid = c * NS + s
        r0 = pl.multiple_of(s * RPT, 8)
        ebase = pl.multiple_of(wid * EPT, 8)

        zv = jnp.zeros((16,), jnp.float32)

        def zrow(i, carry):
            for j in range(D // 16):
                rows0[i, pl.ds(j * 16, 16)] = zv
            return carry

        lax.fori_loop(0, CH, zrow, 0)
        for k in range(ZFULL):
            pltpu.sync_copy(rows0, acc_sh.at[pl.ds(r0 + k * CH, CH)])
        pltpu.sync_copy(rows0.at[pl.ds(0, ZREM)],
                        acc_sh.at[pl.ds(r0 + ZFULL * CH, ZREM)])
        if with_deg:
            for j in range(CH // 16):
                ones_v[pl.ds(j * 16, 16)] = jnp.ones((16,), jnp.float32)
            for j in range(RPT // 16):
                degrow_v[pl.ds(j * 16, 16)] = zv
            pltpu.sync_copy(degrow_v, deg_sh.at[pl.ds(r0, RPT)])

        @pl.when(s == NS - 1)
        def _zero_tail():
            pltpu.sync_copy(rows0.at[pl.ds(0, R_TAIL)],
                            acc_sh.at[pl.ds(R_TAIL0, R_TAIL)])
            if with_deg:
                pltpu.sync_copy(degrow_v.at[pl.ds(0, R_TAIL)],
                                deg_sh.at[pl.ds(R_TAIL0, R_TAIL)])

        plsc.subcore_barrier()

        pltpu.sync_copy(src.at[pl.ds(ebase, EPT)], src_all)

        def src_sl(i):
            return src_all.at[pl.ds(pl.multiple_of(i * CH, 8), CH)]

        def start_chunk(i, q):
            dst_v, rows, isem, gsem = sets[q][0], sets[q][1], sets[q][2], sets[q][3]
            pltpu.async_copy(dst.at[pl.ds(pl.multiple_of(ebase + i * CH, 8), CH)],
                             dst_v, isem)
            pltpu.async_copy(feat.at[src_sl(i)], rows, gsem)

        def start_scatter(q):
            dst_v, rows, ssem, dsem = sets[q][0], sets[q][1], sets[q][4], sets[q][5]
            pltpu.async_copy(rows, acc_sh.at[dst_v], ssem, add=True)
            if with_deg:
                pltpu.async_copy(ones_v, deg_sh.at[dst_v], dsem, add=True)

        def wait_idx(q):
            dst_v, isem = sets[q][0], sets[q][2]
            pltpu.make_async_copy(dst.at[pl.ds(ebase, CH)], dst_v, isem).wait()

        def wait_gather(q):
            rows, gsem = sets[q][1], sets[q][3]
            pltpu.make_async_copy(feat.at[src_sl(0)], rows, gsem).wait()

        def wait_scatter(q):
            dst_v, rows, ssem, dsem = sets[q][0], sets[q][1], sets[q][4], sets[q][5]
            pltpu.make_async_copy(rows, acc_sh.at[dst_v], ssem).wait()
            if with_deg:
                pltpu.make_async_copy(ones_v, deg_sh.at[dst_v], dsem).wait()

        rem_lo = pl.multiple_of(NFULL * CH, 8)
        pltpu.sync_copy(dst.at[pl.ds(ebase + rem_lo, REM_E)], dst_rem)
        pltpu.async_copy(feat.at[src_all.at[pl.ds(rem_lo, REM_E)]],
                         rows0.at[pl.ds(0, REM_E)], sets[0][3]).wait()
        d_rem = pltpu.async_copy(rows0.at[pl.ds(0, REM_E)],
                                 acc_sh.at[dst_rem], sets[0][4], add=True)
        if with_deg:
            pltpu.async_copy(ones_v.at[pl.ds(0, REM_E)],
                             deg_sh.at[dst_rem], sets[0][5], add=True).wait()
        d_rem.wait()
        for t, i in enumerate(range(NMAIN, NFULL)):
            start_chunk(i, t)
        for t, i in enumerate(range(NMAIN, NFULL)):
            wait_gather(t)
            wait_idx(t)
            start_scatter(t)
        for t, i in enumerate(range(NMAIN, NFULL)):
            wait_scatter(t)

        for q in range(NSETS):
            start_chunk(q, q)

        def quad(g, carry):
            i = NSETS * g
            for q in range(NSETS):
                wait_gather(q)
                wait_idx(q)
                start_scatter(q)
            for q in range(NSETS):
                nq = jnp.where(i + q + NSETS < NMAIN, i + q + NSETS, 0)
                wait_scatter(q)
                start_chunk(nq, q)
            return carry

        lax.fori_loop(0, NQUAD, quad, 0)
        for q in range(NSETS):
            wait_idx(q)
            wait_gather(q)
        plsc.subcore_barrier()

        def wr_start(lo, nrows, q):
            rows, ssem = sets[q][1], sets[q][4]
            pltpu.sync_copy(acc_sh.at[pl.ds(lo, nrows)],
                            rows.at[pl.ds(0, nrows)])
            pltpu.async_copy(rows.at[pl.ds(0, nrows)],
                             out.at[c, pl.ds(lo, nrows)], ssem)

        def wr_wait(lo, nrows, q):
            rows, ssem = sets[q][1], sets[q][4]
            pltpu.make_async_copy(rows.at[pl.ds(0, nrows)],
                                  out.at[c, pl.ds(lo, nrows)], ssem).wait()

        lo_rem = pl.multiple_of(r0 + ZFULL * CH, 8)
        outstanding = {}
        for k in range(ZFULL):
            q = k % NSETS
            if q in outstanding:
                wr_wait(lo_rem, outstanding.pop(q), q)
            wr_start(pl.multiple_of(r0 + k * CH, 8), CH, q)
            outstanding[q] = CH
        q = ZFULL % NSETS
        if q in outstanding:
            wr_wait(lo_rem, outstanding.pop(q), q)
        wr_start(lo_rem, ZREM, q)
        outstanding[q] = ZREM
        for q2, nr in outstanding.items():
            wr_wait(lo_rem, nr, q2)
        if with_deg:
            pltpu.sync_copy(deg_sh.at[pl.ds(r0, RPT)], degrow_v)
            d0 = pl.multiple_of(c * N + r0, 8)
            pltpu.sync_copy(degrow_v, deg_out.at[pl.ds(d0, RPT)])

        @pl.when(s == NS - 1)
        def _write_tail():
            pltpu.sync_copy(acc_sh.at[pl.ds(R_TAIL0, R_TAIL)],
                            rows0.at[pl.ds(0, R_TAIL)])
            pltpu.sync_copy(rows0.at[pl.ds(0, R_TAIL)],
                            out.at[c, pl.ds(R_TAIL0, R_TAIL)])
            if with_deg:
                pltpu.sync_copy(deg_sh.at[pl.ds(R_TAIL0, R_TAIL)],
                                degrow_v.at[pl.ds(0, R_TAIL)])
                dt = pl.multiple_of(c * N + R_TAIL0, 8)
                pltpu.sync_copy(degrow_v.at[pl.ds(0, R_TAIL)],
                                deg_out.at[pl.ds(dt, R_TAIL)])

    return pl.kernel(
        body,
        out_type=out_type,
        mesh=plsc.VectorSubcoreMesh(core_axis_name="c", subcore_axis_name="s"),
        scratch_types=scratch,
        name=f"sc_agg_d{D}" + ("_deg" if with_deg else ""),
    )


_sc_agg_deg = _make_sc_agg(F_HID, with_deg=True)
_sc_agg = _make_sc_agg(F_HID, with_deg=False)


_BR = 1000


def _mm1_body(x_ref, ws_ref, wn_ref, b_ref, os_ref, on_ref):
    xb = x_ref[...]
    os_ref[...] = jnp.dot(xb, ws_ref[...], preferred_element_type=jnp.float32) + b_ref[...]
    on_ref[...] = jnp.dot(xb, wn_ref[...], preferred_element_type=jnp.float32)


def _tc_mm1(x, W_self, W_neigh, b):
    return pl.pallas_call(
        _mm1_body,
        grid=(N // _BR,),
        in_specs=[
            pl.BlockSpec((_BR, F_IN), lambda i: (i, 0)),
            pl.BlockSpec((F_IN, F_HID), lambda i: (0, 0)),
            pl.BlockSpec((F_IN, F_HID), lambda i: (0, 0)),
            pl.BlockSpec((1, F_HID), lambda i: (0, 0)),
        ],
        out_specs=[
            pl.BlockSpec((_BR, F_HID), lambda i: (i, 0)),
            pl.BlockSpec((_BR, F_HID), lambda i: (i, 0)),
        ],
        out_shape=[
            jax.ShapeDtypeStruct((N, F_HID), jnp.float32),
            jax.ShapeDtypeStruct((N, F_HID), jnp.float32),
        ],
    )(x, W_self, W_neigh, b.reshape(1, -1))


def _l2_body(xs_ref, agg_ref, deg_ref, ws_ref, b_ref, hs_ref, h_ref):
    deg = jnp.maximum(deg_ref[0] + deg_ref[1], 1.0)
    h = jnp.maximum(xs_ref[...] + (agg_ref[0] + agg_ref[1]) / deg, 0.0)
    h_ref[...] = h
    hs_ref[...] = jnp.dot(h, ws_ref[...], preferred_element_type=jnp.float32) + b_ref[...]


def _tc_layer2(xs, agg, deg2, W_self, b):
    return pl.pallas_call(
        _l2_body,
        grid=(N // _BR,),
        in_specs=[
            pl.BlockSpec((_BR, F_HID), lambda i: (i, 0)),
            pl.BlockSpec((NC, _BR, F_HID), lambda i: (0, i, 0)),
            pl.BlockSpec((NC, _BR, 1), lambda i: (0, i, 0)),
            pl.BlockSpec((F_HID, F_OUT), lambda i: (0, 0)),
            pl.BlockSpec((1, F_OUT), lambda i: (0, 0)),
        ],
        out_specs=[
            pl.BlockSpec((_BR, F_OUT), lambda i: (i, 0)),
            pl.BlockSpec((_BR, F_HID), lambda i: (i, 0)),
        ],
        out_shape=[
            jax.ShapeDtypeStruct((N, F_OUT), jnp.float32),
            jax.ShapeDtypeStruct((N, F_HID), jnp.float32),
        ],
    )(xs, agg, deg2, W_self, b.reshape(1, -1))


def _comb_body(hs_ref, agg_ref, deg_ref, wn_ref, out_ref):
    deg = jnp.maximum(deg_ref[0] + deg_ref[1], 1.0)
    h_neigh = (agg_ref[0] + agg_ref[1]) / deg
    out_ref[...] = hs_ref[...] + jnp.dot(
        h_neigh, wn_ref[...], preferred_element_type=jnp.float32)


def _tc_combine(hs, agg, deg2, W_neigh):
    return pl.pallas_call(
        _comb_body,
        grid=(N // _BR,),
        in_specs=[
            pl.BlockSpec((_BR, F_OUT), lambda i: (i, 0)),
            pl.BlockSpec((NC, _BR, F_HID), lambda i: (0, i, 0)),
            pl.BlockSpec((NC, _BR, 1), lambda i: (0, i, 0)),
            pl.BlockSpec((F_HID, F_OUT), lambda i: (0, 0)),
        ],
        out_specs=pl.BlockSpec((_BR, F_OUT), lambda i: (i, 0)),
        out_shape=jax.ShapeDtypeStruct((N, F_OUT), jnp.float32),
    )(hs, agg, deg2, W_neigh)


def kernel(x, edge_index, W_self1, W_neigh1, b1, W_self2, W_neigh2, b2):
    src = edge_index[0].astype(jnp.int32)
    dst = edge_index[1].astype(jnp.int32)

    xs1, xn1 = _tc_mm1(x, W_self1, W_neigh1, b1)
    agg1, deg = _sc_agg_deg(xn1, src, dst)
    deg2 = deg.reshape(NC, N, 1)
    hs2, h = _tc_layer2(xs1, agg1, deg2, W_self2, b2)
    agg2 = _sc_agg(h, src, dst)
    return _tc_combine(hs2, agg2, deg2, W_neigh2)


# speedup vs baseline: 12.2346x; 1.0409x over previous
import jax
import jax.numpy as jnp
from jax import lax
from jax.experimental import pallas as pl
from jax.experimental.pallas import tpu as pltpu
from jax.experimental.pallas import tpu_sc as plsc

N = 10000
E = 320000
F_IN = 128
F_HID = 128
F_OUT = 64

NC = 2
NS = 16
NW = NC * NS
EPT = E // NW
CH = 64
NFULL = EPT // CH
REM_E = EPT - NFULL * CH
NSETS = 4
NQUAD = NFULL // NSETS
NMAIN = NQUAD * NSETS
RPT = 624
R_TAIL0 = NS * RPT
R_TAIL = N - R_TAIL0
ZFULL = RPT // CH
ZREM = RPT - ZFULL * CH


def _make_sc_agg(D, with_deg):
    out_type = jax.ShapeDtypeStruct((NC, N, D), jnp.float32)
    if with_deg:
        out_type = (out_type, jax.ShapeDtypeStruct((NC * N,), jnp.float32))
    scratch = [
        pltpu.VMEM_SHARED((N, D), jnp.float32),
        pltpu.VMEM((EPT,), jnp.int32),
        pltpu.VMEM((REM_E,), jnp.int32),
    ]
    for _ in range(NSETS):
        scratch += [
            pltpu.VMEM((CH,), jnp.int32),
            pltpu.VMEM((CH, D), jnp.float32),
            pltpu.SemaphoreType.DMA,
            pltpu.SemaphoreType.DMA,
            pltpu.SemaphoreType.DMA,
            pltpu.SemaphoreType.DMA,
        ]
    if with_deg:
        scratch += [
            pltpu.VMEM_SHARED((N,), jnp.float32),
            pltpu.VMEM((CH,), jnp.float32),
            pltpu.VMEM((RPT,), jnp.float32),
        ]

    def body(feat, src, dst, *refs):
        if with_deg:
            out, deg_out = refs[0], refs[1]
            rest = refs[2:]
        else:
            out = refs[0]
            rest = refs[1:]
        acc_sh, src_all, dst_rem = rest[0], rest[1], rest[2]
        sets = [tuple(rest[3 + 6 * q: 3 + 6 * (q + 1)]) for q in range(NSETS)]
        if with_deg:
            deg_sh, ones_v, degrow_v = rest[3 + 6 * NSETS:]
        rows0 = sets[0][1]
        c = lax.axis_index("c")
        s = lax.axis_index("s")
        wid = c * NS + s
        r0 = pl.multiple_of(s * RPT, 8)
        ebase = pl.multiple_of(wid * EPT, 8)

        zv = jnp.zeros((16,), jnp.float32)

        def zrow(i, carry):
            for j in range(D // 16):
                rows0[i, pl.ds(j * 16, 16)] = zv
            return carry

        lax.fori_loop(0, CH, zrow, 0)
        for k in range(ZFULL):
            pltpu.sync_copy(rows0, acc_sh.at[pl.ds(r0 + k * CH, CH)])
        pltpu.sync_copy(rows0.at[pl.ds(0, ZREM)],
                        acc_sh.at[pl.ds(r0 + ZFULL * CH, ZREM)])
        if with_deg:
            for j in range(CH // 16):
                ones_v[pl.ds(j * 16, 16)] = jnp.ones((16,), jnp.float32)
            for j in range(RPT // 16):
                degrow_v[pl.ds(j * 16, 16)] = zv
            pltpu.sync_copy(degrow_v, deg_sh.at[pl.ds(r0, RPT)])

        @pl.when(s == NS - 1)
        def _zero_tail():
            pltpu.sync_copy(rows0.at[pl.ds(0, R_TAIL)],
                            acc_sh.at[pl.ds(R_TAIL0, R_TAIL)])
            if with_deg:
                pltpu.sync_copy(degrow_v.at[pl.ds(0, R_TAIL)],
                                deg_sh.at[pl.ds(R_TAIL0, R_TAIL)])

        plsc.subcore_barrier()

        pltpu.sync_copy(src.at[pl.ds(ebase, EPT)], src_all)

        def src_sl(i):
            return src_all.at[pl.ds(pl.multiple_of(i * CH, 8), CH)]

        def start_chunk(i, q):
            dst_v, rows, isem, gsem = sets[q][0], sets[q][1], sets[q][2], sets[q][3]
            pltpu.async_copy(dst.at[pl.ds(pl.multiple_of(ebase + i * CH, 8), CH)],
                             dst_v, isem)
            pltpu.async_copy(feat.at[src_sl(i)], rows, gsem)

        def start_scatter(q):
            dst_v, rows, ssem, dsem = sets[q][0], sets[q][1], sets[q][4], sets[q][5]
            pltpu.async_copy(rows, acc_sh.at[dst_v], ssem, add=True)
            if with_deg:
                pltpu.async_copy(ones_v, deg_sh.at[dst_v], dsem, add=True)

        def wait_idx(q):
            dst_v, isem = sets[q][0], sets[q][2]
            pltpu.make_async_copy(dst.at[pl.ds(ebase, CH)], dst_v, isem).wait()

        def wait_gather(q):
            rows, gsem = sets[q][1], sets[q][3]
            pltpu.make_async_copy(feat.at[src_sl(0)], rows, gsem).wait()

        def wait_scatter(q):
            dst_v, rows, ssem, dsem = sets[q][0], sets[q][1], sets[q][4], sets[q][5]
            pltpu.make_async_copy(rows, acc_sh.at[dst_v], ssem).wait()
            if with_deg:
                pltpu.make_async_copy(ones_v, deg_sh.at[dst_v], dsem).wait()

        rem_lo = pl.multiple_of(NFULL * CH, 8)
        pltpu.sync_copy(dst.at[pl.ds(ebase + rem_lo, REM_E)], dst_rem)
        pltpu.async_copy(feat.at[src_all.at[pl.ds(rem_lo, REM_E)]],
                         rows0.at[pl.ds(0, REM_E)], sets[0][3]).wait()
        d_rem = pltpu.async_copy(rows0.at[pl.ds(0, REM_E)],
                                 acc_sh.at[dst_rem], sets[0][4], add=True)
        if with_deg:
            pltpu.async_copy(ones_v.at[pl.ds(0, REM_E)],
                             deg_sh.at[dst_rem], sets[0][5], add=True).wait()
        d_rem.wait()
        for t, i in enumerate(range(NMAIN, NFULL)):
            start_chunk(i, t)
        for t, i in enumerate(range(NMAIN, NFULL)):
            wait_gather(t)
            wait_idx(t)
            start_scatter(t)
        for t, i in enumerate(range(NMAIN, NFULL)):
            wait_scatter(t)

        for q in range(NSETS):
            start_chunk(q, q)

        def quad(g, carry):
            i = NSETS * g
            for q in range(NSETS):
                wait_gather(q)
                wait_idx(q)
                start_scatter(q)
            for q in range(NSETS):
                nq = jnp.where(i + q + NSETS < NMAIN, i + q + NSETS, 0)
                wait_scatter(q)
                start_chunk(nq, q)
            return carry

        lax.fori_loop(0, NQUAD, quad, 0)
        for q in range(NSETS):
            wait_idx(q)
            wait_gather(q)
        plsc.subcore_barrier()

        def wr_start(lo, nrows, q):
            rows, ssem = sets[q][1], sets[q][4]
            pltpu.sync_copy(acc_sh.at[pl.ds(lo, nrows)],
                            rows.at[pl.ds(0, nrows)])
            pltpu.async_copy(rows.at[pl.ds(0, nrows)],
                             out.at[c, pl.ds(lo, nrows)], ssem)

        def wr_wait(lo, nrows, q):
            rows, ssem = sets[q][1], sets[q][4]
            pltpu.make_async_copy(rows.at[pl.ds(0, nrows)],
                                  out.at[c, pl.ds(lo, nrows)], ssem).wait()

        lo_rem = pl.multiple_of(r0 + ZFULL * CH, 8)
        outstanding = {}
        for k in range(ZFULL):
            q = k % NSETS
            if q in outstanding:
                wr_wait(lo_rem, outstanding.pop(q), q)
            wr_start(pl.multiple_of(r0 + k * CH, 8), CH, q)
            outstanding[q] = CH
        q = ZFULL % NSETS
        if q in outstanding:
            wr_wait(lo_rem, outstanding.pop(q), q)
        wr_start(lo_rem, ZREM, q)
        outstanding[q] = ZREM
        for q2, nr in outstanding.items():
            wr_wait(lo_rem, nr, q2)
        if with_deg:
            pltpu.sync_copy(deg_sh.at[pl.ds(r0, RPT)], degrow_v)
            d0 = pl.multiple_of(c * N + r0, 8)
            pltpu.sync_copy(degrow_v, deg_out.at[pl.ds(d0, RPT)])

        @pl.when(s == NS - 1)
        def _write_tail():
            pltpu.sync_copy(acc_sh.at[pl.ds(R_TAIL0, R_TAIL)],
                            rows0.at[pl.ds(0, R_TAIL)])
            pltpu.sync_copy(rows0.at[pl.ds(0, R_TAIL)],
                            out.at[c, pl.ds(R_TAIL0, R_TAIL)])
            if with_deg:
                pltpu.sync_copy(deg_sh.at[pl.ds(R_TAIL0, R_TAIL)],
                                degrow_v.at[pl.ds(0, R_TAIL)])
                dt = pl.multiple_of(c * N + R_TAIL0, 8)
                pltpu.sync_copy(degrow_v.at[pl.ds(0, R_TAIL)],
                                deg_out.at[pl.ds(dt, R_TAIL)])

    return pl.kernel(
        body,
        out_type=out_type,
        mesh=plsc.VectorSubcoreMesh(core_axis_name="c", subcore_axis_name="s"),
        scratch_types=scratch,
        name=f"sc_agg_d{D}" + ("_deg" if with_deg else ""),
    )


_sc_agg_deg = _make_sc_agg(F_HID, with_deg=True)
_sc_agg = _make_sc_agg(F_HID, with_deg=False)


_BR = 1000


def _mm1_body(x_ref, ws_ref, wn_ref, b_ref, os_ref, on_ref):
    xb = x_ref[...]
    os_ref[...] = jnp.dot(xb, ws_ref[...], preferred_element_type=jnp.float32) + b_ref[...]
    on_ref[...] = jnp.dot(xb, wn_ref[...], preferred_element_type=jnp.float32)


def _tc_mm1(x, W_self, W_neigh, b):
    return pl.pallas_call(
        _mm1_body,
        grid=(N // _BR,),
        in_specs=[
            pl.BlockSpec((_BR, F_IN), lambda i: (i, 0)),
            pl.BlockSpec((F_IN, F_HID), lambda i: (0, 0)),
            pl.BlockSpec((F_IN, F_HID), lambda i: (0, 0)),
            pl.BlockSpec((1, F_HID), lambda i: (0, 0)),
        ],
        out_specs=[
            pl.BlockSpec((_BR, F_HID), lambda i: (i, 0)),
            pl.BlockSpec((_BR, F_HID), lambda i: (i, 0)),
        ],
        out_shape=[
            jax.ShapeDtypeStruct((N, F_HID), jnp.float32),
            jax.ShapeDtypeStruct((N, F_HID), jnp.float32),
        ],
    )(x, W_self, W_neigh, b.reshape(1, -1))


def _l2_body(xs_ref, agg_ref, deg_ref, ws_ref, b_ref, hs_ref, h_ref):
    deg = jnp.maximum(deg_ref[0] + deg_ref[1], 1.0)
    h = jnp.maximum(xs_ref[...] + (agg_ref[0] + agg_ref[1]) / deg, 0.0)
    h_ref[...] = h
    hs_ref[...] = jnp.dot(h, ws_ref[...], preferred_element_type=jnp.float32) + b_ref[...]


def _tc_layer2(xs, agg, deg2, W_self, b):
    return pl.pallas_call(
        _l2_body,
        grid=(N // _BR,),
        in_specs=[
            pl.BlockSpec((_BR, F_HID), lambda i: (i, 0)),
            pl.BlockSpec((NC, _BR, F_HID), lambda i: (0, i, 0)),
            pl.BlockSpec((NC, _BR, 1), lambda i: (0, i, 0)),
            pl.BlockSpec((F_HID, F_OUT), lambda i: (0, 0)),
            pl.BlockSpec((1, F_OUT), lambda i: (0, 0)),
        ],
        out_specs=[
            pl.BlockSpec((_BR, F_OUT), lambda i: (i, 0)),
            pl.BlockSpec((_BR, F_HID), lambda i: (i, 0)),
        ],
        out_shape=[
            jax.ShapeDtypeStruct((N, F_OUT), jnp.float32),
            jax.ShapeDtypeStruct((N, F_HID), jnp.float32),
        ],
    )(xs, agg, deg2, W_self, b.reshape(1, -1))


def _comb_body(hs_ref, agg_ref, deg_ref, wn_ref, out_ref):
    deg = jnp.maximum(deg_ref[0] + deg_ref[1], 1.0)
    h_neigh = (agg_ref[0] + agg_ref[1]) / deg
    out_ref[...] = hs_ref[...] + jnp.dot(
        h_neigh, wn_ref[...], preferred_element_type=jnp.float32)


def _tc_combine(hs, agg, deg2, W_neigh):
    return pl.pallas_call(
        _comb_body,
        grid=(N // _BR,),
        in_specs=[
            pl.BlockSpec((_BR, F_OUT), lambda i: (i, 0)),
            pl.BlockSpec((NC, _BR, F_HID), lambda i: (0, i, 0)),
            pl.BlockSpec((NC, _BR, 1), lambda i: (0, i, 0)),
            pl.BlockSpec((F_HID, F_OUT), lambda i: (0, 0)),
        ],
        out_specs=pl.BlockSpec((_BR, F_OUT), lambda i: (i, 0)),
        out_shape=jax.ShapeDtypeStruct((N, F_OUT), jnp.float32),
    )(hs, agg, deg2, W_neigh)


def kernel(x, edge_index, W_self1, W_neigh1, b1, W_self2, W_neigh2, b2):
    src = edge_index[0].astype(jnp.int32)
    dst = edge_index[1].astype(jnp.int32)

    xs1, xn1 = _tc_mm1(x, W_self1, W_neigh1, b1)
    agg1, deg = _sc_agg_deg(xn1, src, dst)
    deg2 = deg.reshape(NC, N, 1)
    hs2, h = _tc_layer2(xs1, agg1, deg2, W_self2, b2)
    agg2 = _sc_agg(h, src, dst)
    return _tc_combine(hs2, agg2, deg2, W_neigh2)


# speedup vs baseline: 12.3443x; 1.0090x over previous
import jax
import jax.numpy as jnp
from jax import lax
from jax.experimental import pallas as pl
from jax.experimental.pallas import tpu as pltpu
from jax.experimental.pallas import tpu_sc as plsc

N = 10000
E = 320000
F_IN = 128
F_HID = 128
F_OUT = 64

NC = 2
NS = 16
NW = NC * NS
EPT = E // NW
CH = 64
NFULL = EPT // CH
REM_E = EPT - NFULL * CH
NSETS = 4
NQUAD = NFULL // NSETS
NMAIN = NQUAD * NSETS
RPT = 624
R_TAIL0 = NS * RPT
R_TAIL = N - R_TAIL0
ZFULL = RPT // CH
ZREM = RPT - ZFULL * CH


def _make_sc_agg(D, with_deg):
    out_type = jax.ShapeDtypeStruct((NC, N, D), jnp.float32)
    if with_deg:
        out_type = (out_type, jax.ShapeDtypeStruct((NC * N,), jnp.float32))
    scratch = [
        pltpu.VMEM_SHARED((N, D), jnp.float32),
        pltpu.VMEM((EPT,), jnp.int32),
        pltpu.VMEM((REM_E,), jnp.int32),
    ]
    for _ in range(NSETS):
        scratch += [
            pltpu.VMEM((CH,), jnp.int32),
            pltpu.VMEM((CH, D), jnp.float32),
            pltpu.SemaphoreType.DMA,
            pltpu.SemaphoreType.DMA,
            pltpu.SemaphoreType.DMA,
            pltpu.SemaphoreType.DMA,
        ]
    if with_deg:
        scratch += [
            pltpu.VMEM_SHARED((N,), jnp.float32),
            pltpu.VMEM((CH,), jnp.float32),
            pltpu.VMEM((RPT,), jnp.float32),
        ]

    def body(feat, src, dst, *refs):
        if with_deg:
            out, deg_out = refs[0], refs[1]
            rest = refs[2:]
        else:
            out = refs[0]
            rest = refs[1:]
        acc_sh, src_all, dst_rem = rest[0], rest[1], rest[2]
        sets = [tuple(rest[3 + 6 * q: 3 + 6 * (q + 1)]) for q in range(NSETS)]
        if with_deg:
            deg_sh, ones_v, degrow_v = rest[3 + 6 * NSETS:]
        rows0 = sets[0][1]
        c = lax.axis_index("c")
        s = lax.axis_index("s")
        wid = c * NS + s
        r0 = pl.multiple_of(s * RPT, 8)
        ebase = pl.multiple_of(wid * EPT, 8)

        zv = jnp.zeros((16,), jnp.float32)

        def zrow(i, carry):
            for j in range(D // 16):
                rows0[i, pl.ds(j * 16, 16)] = zv
            return carry

        lax.fori_loop(0, CH, zrow, 0)
        for k in range(ZFULL):
            pltpu.sync_copy(rows0, acc_sh.at[pl.ds(r0 + k * CH, CH)])
        pltpu.sync_copy(rows0.at[pl.ds(0, ZREM)],
                        acc_sh.at[pl.ds(r0 + ZFULL * CH, ZREM)])
        if with_deg:
            for j in range(CH // 16):
                ones_v[pl.ds(j * 16, 16)] = jnp.ones((16,), jnp.float32)
            for j in range(RPT // 16):
                degrow_v[pl.ds(j * 16, 16)] = zv
            pltpu.sync_copy(degrow_v, deg_sh.at[pl.ds(r0, RPT)])

        @pl.when(s == NS - 1)
        def _zero_tail():
            pltpu.sync_copy(rows0.at[pl.ds(0, R_TAIL)],
                            acc_sh.at[pl.ds(R_TAIL0, R_TAIL)])
            if with_deg:
                pltpu.sync_copy(degrow_v.at[pl.ds(0, R_TAIL)],
                                deg_sh.at[pl.ds(R_TAIL0, R_TAIL)])

        plsc.subcore_barrier()

        pltpu.sync_copy(src.at[pl.ds(ebase, EPT)], src_all)

        def src_sl(i):
            return src_all.at[pl.ds(pl.multiple_of(i * CH, 8), CH)]

        def start_chunk(i, q):
            dst_v, rows, isem, gsem = sets[q][0], sets[q][1], sets[q][2], sets[q][3]
            pltpu.async_copy(dst.at[pl.ds(pl.multiple_of(ebase + i * CH, 8), CH)],
                             dst_v, isem)
            pltpu.async_copy(feat.at[src_sl(i)], rows, gsem)

        def start_scatter(q):
            dst_v, rows, ssem, dsem = sets[q][0], sets[q][1], sets[q][4], sets[q][5]
            pltpu.async_copy(rows, acc_sh.at[dst_v], ssem, add=True)
            if with_deg:
                pltpu.async_copy(ones_v, deg_sh.at[dst_v], dsem, add=True)

        def wait_idx(q):
            dst_v, isem = sets[q][0], sets[q][2]
            pltpu.make_async_copy(dst.at[pl.ds(ebase, CH)], dst_v, isem).wait()

        def wait_gather(q):
            rows, gsem = sets[q][1], sets[q][3]
            pltpu.make_async_copy(feat.at[src_sl(0)], rows, gsem).wait()

        def wait_scatter(q):
            dst_v, rows, ssem, dsem = sets[q][0], sets[q][1], sets[q][4], sets[q][5]
            pltpu.make_async_copy(rows, acc_sh.at[dst_v], ssem).wait()
            if with_deg:
                pltpu.make_async_copy(ones_v, deg_sh.at[dst_v], dsem).wait()

        rem_lo = pl.multiple_of(NFULL * CH, 8)
        pltpu.sync_copy(dst.at[pl.ds(ebase + rem_lo, REM_E)], dst_rem)
        pltpu.async_copy(feat.at[src_all.at[pl.ds(rem_lo, REM_E)]],
                         rows0.at[pl.ds(0, REM_E)], sets[0][3]).wait()
        d_rem = pltpu.async_copy(rows0.at[pl.ds(0, REM_E)],
                                 acc_sh.at[dst_rem], sets[0][4], add=True)
        if with_deg:
            pltpu.async_copy(ones_v.at[pl.ds(0, REM_E)],
                             deg_sh.at[dst_rem], sets[0][5], add=True).wait()
        d_rem.wait()
        for t, i in enumerate(range(NMAIN, NFULL)):
            start_chunk(i, t)
        for t, i in enumerate(range(NMAIN, NFULL)):
            wait_gather(t)
            wait_idx(t)
            start_scatter(t)
        for t, i in enumerate(range(NMAIN, NFULL)):
            wait_scatter(t)

        for q in range(NSETS):
            start_chunk(q, q)

        def quad(g, carry):
            i = NSETS * g
            for q in range(NSETS):
                wait_gather(q)
                wait_idx(q)
                start_scatter(q)
            for q in range(NSETS):
                nq = jnp.where(i + q + NSETS < NMAIN, i + q + NSETS, 0)
                wait_scatter(q)
                start_chunk(nq, q)
            return carry

        lax.fori_loop(0, NQUAD, quad, 0)
        for q in range(NSETS):
            wait_idx(q)
            wait_gather(q)
        plsc.subcore_barrier()

        def wr_start(lo, nrows, q):
            rows, ssem = sets[q][1], sets[q][4]
            pltpu.sync_copy(acc_sh.at[pl.ds(lo, nrows)],
                            rows.at[pl.ds(0, nrows)])
            pltpu.async_copy(rows.at[pl.ds(0, nrows)],
                             out.at[c, pl.ds(lo, nrows)], ssem)

        def wr_wait(lo, nrows, q):
            rows, ssem = sets[q][1], sets[q][4]
            pltpu.make_async_copy(rows.at[pl.ds(0, nrows)],
                                  out.at[c, pl.ds(lo, nrows)], ssem).wait()

        lo_rem = pl.multiple_of(r0 + ZFULL * CH, 8)
        outstanding = {}
        for k in range(ZFULL):
            q = k % NSETS
            if q in outstanding:
                wr_wait(lo_rem, outstanding.pop(q), q)
            wr_start(pl.multiple_of(r0 + k * CH, 8), CH, q)
            outstanding[q] = CH
        q = ZFULL % NSETS
        if q in outstanding:
            wr_wait(lo_rem, outstanding.pop(q), q)
        wr_start(lo_rem, ZREM, q)
        outstanding[q] = ZREM
        for q2, nr in outstanding.items():
            wr_wait(lo_rem, nr, q2)
        if with_deg:
            pltpu.sync_copy(deg_sh.at[pl.ds(r0, RPT)], degrow_v)
            d0 = pl.multiple_of(c * N + r0, 8)
            pltpu.sync_copy(degrow_v, deg_out.at[pl.ds(d0, RPT)])

        @pl.when(s == NS - 1)
        def _write_tail():
            pltpu.sync_copy(acc_sh.at[pl.ds(R_TAIL0, R_TAIL)],
                            rows0.at[pl.ds(0, R_TAIL)])
            pltpu.sync_copy(rows0.at[pl.ds(0, R_TAIL)],
                            out.at[c, pl.ds(R_TAIL0, R_TAIL)])
            if with_deg:
                pltpu.sync_copy(deg_sh.at[pl.ds(R_TAIL0, R_TAIL)],
                                degrow_v.at[pl.ds(0, R_TAIL)])
                dt = pl.multiple_of(c * N + R_TAIL0, 8)
                pltpu.sync_copy(degrow_v.at[pl.ds(0, R_TAIL)],
                                deg_out.at[pl.ds(dt, R_TAIL)])

    return pl.kernel(
        body,
        out_type=out_type,
        mesh=plsc.VectorSubcoreMesh(core_axis_name="c", subcore_axis_name="s"),
        scratch_types=scratch,
        name=f"sc_agg_d{D}" + ("_deg" if with_deg else ""),
    )


_sc_agg_deg = _make_sc_agg(F_HID, with_deg=True)
_sc_agg = _make_sc_agg(F_HID, with_deg=False)


_BR = 1000


def _mm_body(x_ref, w_ref, b_ref, o_ref):
    o_ref[...] = jnp.dot(x_ref[...], w_ref[...],
                         preferred_element_type=jnp.float32) + b_ref[...]


def _tc_mm(x, W, b):
    Din, Dout = W.shape
    return pl.pallas_call(
        _mm_body,
        grid=(N // _BR,),
        in_specs=[
            pl.BlockSpec((_BR, Din), lambda i: (i, 0)),
            pl.BlockSpec((Din, Dout), lambda i: (0, 0)),
            pl.BlockSpec((1, Dout), lambda i: (0, 0)),
        ],
        out_specs=pl.BlockSpec((_BR, Dout), lambda i: (i, 0)),
        out_shape=jax.ShapeDtypeStruct((N, Dout), jnp.float32),
    )(x, W, b.reshape(1, -1))


def _relu_comb_body(xs_ref, agg_ref, deg_ref, h_ref):
    deg = jnp.maximum(deg_ref[0] + deg_ref[1], 1.0)
    h_ref[...] = jnp.maximum(xs_ref[...] + (agg_ref[0] + agg_ref[1]) / deg, 0.0)


def _tc_relu_comb(xs, agg, deg2):
    return pl.pallas_call(
        _relu_comb_body,
        grid=(N // _BR,),
        in_specs=[
            pl.BlockSpec((_BR, F_HID), lambda i: (i, 0)),
            pl.BlockSpec((NC, _BR, F_HID), lambda i: (0, i, 0)),
            pl.BlockSpec((NC, _BR, 1), lambda i: (0, i, 0)),
        ],
        out_specs=pl.BlockSpec((_BR, F_HID), lambda i: (i, 0)),
        out_shape=jax.ShapeDtypeStruct((N, F_HID), jnp.float32),
    )(xs, agg, deg2)


def _comb_body(hs_ref, agg_ref, deg_ref, wn_ref, out_ref):
    deg = jnp.maximum(deg_ref[0] + deg_ref[1], 1.0)
    h_neigh = (agg_ref[0] + agg_ref[1]) / deg
    out_ref[...] = hs_ref[...] + jnp.dot(
        h_neigh, wn_ref[...], preferred_element_type=jnp.float32)


def _tc_combine(hs, agg, deg2, W_neigh):
    return pl.pallas_call(
        _comb_body,
        grid=(N // _BR,),
        in_specs=[
            pl.BlockSpec((_BR, F_OUT), lambda i: (i, 0)),
            pl.BlockSpec((NC, _BR, F_HID), lambda i: (0, i, 0)),
            pl.BlockSpec((NC, _BR, 1), lambda i: (0, i, 0)),
            pl.BlockSpec((F_HID, F_OUT), lambda i: (0, 0)),
        ],
        out_specs=pl.BlockSpec((_BR, F_OUT), lambda i: (i, 0)),
        out_shape=jax.ShapeDtypeStruct((N, F_OUT), jnp.float32),
    )(hs, agg, deg2, W_neigh)


def kernel(x, edge_index, W_self1, W_neigh1, b1, W_self2, W_neigh2, b2):
    src = edge_index[0].astype(jnp.int32)
    dst = edge_index[1].astype(jnp.int32)

    zb1 = jnp.zeros((F_HID,), jnp.float32)
    xn1 = _tc_mm(x, W_neigh1, zb1)
    agg1, deg = _sc_agg_deg(xn1, src, dst)
    xs1 = _tc_mm(x, W_self1, b1)
    deg2 = deg.reshape(NC, N, 1)
    h = _tc_relu_comb(xs1, agg1, deg2)
    agg2 = _sc_agg(h, src, dst)
    hs2 = _tc_mm(h, W_self2, b2)
    return _tc_combine(hs2, agg2, deg2, W_neigh2)
